# Initial kernel scaffold; baseline (speedup 1.0000x reference)
import jax
import jax.numpy as jnp
from jax.experimental import pallas as pl


def kernel(x_user, weight):
    raise NotImplementedError("write your pallas kernel here")


# speedup vs baseline: 94.2867x; 94.2867x over previous
import functools

import jax
import jax.numpy as jnp
from jax import lax
from jax.experimental import pallas as pl
from jax.experimental.pallas import tpu as pltpu
from jax.experimental.pallas import tpu_sc as plsc


def kernel(x_user, weight):
    B, L = x_user.shape
    V, D = weight.shape

    info = plsc.get_sparse_core_info()
    NC, NS, LANES = info.num_cores, info.num_subcores, info.num_lanes
    NW = NC * NS
    b_per_w = B // NW
    n_groups = b_per_w // LANES
    inv_l = jnp.float32(1.0 / L)

    mesh = plsc.VectorSubcoreMesh(core_axis_name="c", subcore_axis_name="s")

    @functools.partial(
        pl.kernel,
        mesh=mesh,
        out_type=jax.ShapeDtypeStruct((B * D,), jnp.float32),
        scratch_types=[
            pltpu.VMEM((b_per_w * L,), jnp.int32),
            pltpu.VMEM((V * D,), jnp.float32),
            pltpu.VMEM((b_per_w * D,), jnp.float32),
        ],
        compiler_params=pltpu.CompilerParams(needs_layout_passes=False),
    )
    def _embbag(x_hbm, w_hbm, out_hbm, idx_v, w_v, out_v):
        wid = lax.axis_index("s") * NC + lax.axis_index("c")
        base = wid * b_per_w
        pltpu.sync_copy(x_hbm.at[pl.ds(base * L, b_per_w * L)], idx_v)
        pltpu.sync_copy(w_hbm, w_v)

        lanes = lax.iota(jnp.int32, LANES)

        def g_body(g, carry):
            rows = g * LANES + lanes
            idx_base = rows * L

            def l_body(l, accs):
                idx16 = plsc.load_gather(idx_v, [idx_base + l])
                waddr = idx16 * D
                return tuple(
                    accs[d] + plsc.load_gather(w_v, [waddr + d])
                    for d in range(D)
                )

            acc0 = tuple(jnp.zeros((LANES,), jnp.float32) for _ in range(D))
            accs = lax.fori_loop(0, L, l_body, acc0)
            out_base = rows * D
            for d in range(D):
                plsc.store_scatter(out_v, [out_base + d], accs[d] * inv_l)
            return carry

        lax.fori_loop(0, n_groups, g_body, 0)
        pltpu.sync_copy(out_v, out_hbm.at[pl.ds(base * D, b_per_w * D)])

    out = _embbag(x_user.reshape(B * L), weight.reshape(V * D))
    return out.reshape(B, D)
